# Initial kernel scaffold; baseline (speedup 1.0000x reference)
import jax
import jax.numpy as jnp
from jax.experimental import pallas as pl


def kernel(x, edge_index, edge_attr, u, batch, W1, b1, gamma, beta, W2, b2):
    raise NotImplementedError("write your pallas kernel here")


# speedup vs baseline: 12.0026x; 12.0026x over previous
import jax
import jax.numpy as jnp
from jax import lax
from jax.experimental import pallas as pl
from jax.experimental.pallas import tpu as pltpu

N = 10000
G = 64
H = 128
BLK = 1000
NB = N // BLK


def _fused_body(batch_ref, x_ref, W1_ref, b1_ref, gamma_ref, beta_ref,
                W2_ref, b2_ref, out_ref, acc_ref, cnt_ref):
    k = pl.program_id(0)

    @pl.when(k == 0)
    def _init():
        acc_ref[...] = jnp.zeros_like(acc_ref)
        cnt_ref[...] = jnp.zeros_like(cnt_ref)

    ids = batch_ref[0, 0, :]
    oh = (ids[:, None] == lax.broadcasted_iota(jnp.int32, (BLK, G), 1))
    oh = oh.astype(jnp.float32)
    xb = x_ref[...]
    acc_ref[...] += lax.dot_general(
        oh, xb, dimension_numbers=(((0,), (0,)), ((), ())),
        preferred_element_type=jnp.float32)
    cnt_ref[...] += lax.dot_general(
        oh, jnp.ones((BLK, H), jnp.float32),
        dimension_numbers=(((0,), (0,)), ((), ())),
        preferred_element_type=jnp.float32)

    @pl.when(k == NB - 1)
    def _finish():
        mean = acc_ref[...] / jnp.maximum(cnt_ref[...], 1.0)
        h = lax.dot_general(mean, W1_ref[...],
                            dimension_numbers=(((1,), (0,)), ((), ())),
                            preferred_element_type=jnp.float32)
        h = h + b1_ref[...]
        mu = jnp.mean(h, axis=0, keepdims=True)
        var = jnp.mean((h - mu) ** 2, axis=0, keepdims=True)
        h = (h - mu) * lax.rsqrt(var + 1e-5) * gamma_ref[...] + beta_ref[...]
        h = jnp.maximum(h, 0.0)
        out = lax.dot_general(h, W2_ref[...],
                              dimension_numbers=(((1,), (0,)), ((), ())),
                              preferred_element_type=jnp.float32)
        out_ref[...] = out + b2_ref[...]


def kernel(x, edge_index, edge_attr, u, batch, W1, b1, gamma, beta, W2, b2):
    del edge_index, edge_attr, u
    batch3 = batch.astype(jnp.int32).reshape(NB, 1, BLK)
    full = lambda shape: pl.BlockSpec(shape, lambda k: (0,) * len(shape))
    out = pl.pallas_call(
        _fused_body,
        grid=(NB,),
        in_specs=[
            pl.BlockSpec((1, 1, BLK), lambda k: (k, 0, 0)),
            pl.BlockSpec((BLK, H), lambda k: (k, 0)),
            full((H, H)),
            full((1, H)),
            full((1, H)),
            full((1, H)),
            full((H, H)),
            full((1, H)),
        ],
        out_specs=pl.BlockSpec((G, H), lambda k: (0, 0)),
        out_shape=jax.ShapeDtypeStruct((G, H), jnp.float32),
        scratch_shapes=[
            pltpu.VMEM((G, H), jnp.float32),
            pltpu.VMEM((G, H), jnp.float32),
        ],
    )(batch3, x, W1, b1.reshape(1, H), gamma.reshape(1, H),
      beta.reshape(1, H), W2, b2.reshape(1, H))
    return out
